# Initial kernel scaffold; baseline (speedup 1.0000x reference)
import functools

import jax
import jax.numpy as jnp
from jax import lax
from jax.experimental import pallas as pl
from jax.experimental.pallas import tpu as pltpu
from jax.experimental.pallas import tpu_sc as plsc

VOCAB = 1000
B, T = 1024, 50
N = B * T
NC, NS = 2, 16
NW = NC * NS
PER_W = N // NW
CHUNK = 32
NCHUNK = PER_W // CHUNK


def _lse_body(table_ref, lse_ref):
    x = table_ref[...]
    m = jnp.max(x, axis=1, keepdims=True)
    s = jnp.sum(jnp.exp(x - m), axis=1, keepdims=True)
    lse_ref[...] = m + jnp.log(s)


def _row_lse(table):
    return pl.pallas_call(
        _lse_body,
        out_shape=jax.ShapeDtypeStruct((VOCAB, 1), jnp.float32),
    )(table)


def _fin_body(p_ref, loss_ref):
    loss_ref[0, 0] = jnp.sum(p_ref[...]) * (1.0 / N)


def _finalize(partials):
    return pl.pallas_call(
        _fin_body,
        out_shape=jax.ShapeDtypeStruct((1, 1), jnp.float32),
    )(partials)


_mesh = plsc.VectorSubcoreMesh(core_axis_name="c", subcore_axis_name="s")


@functools.partial(
    pl.kernel,
    out_type=(
        jax.ShapeDtypeStruct((N, VOCAB), jnp.float32),
        jax.ShapeDtypeStruct((NW, 16), jnp.float32),
    ),
    mesh=_mesh,
    scratch_types=(
        pltpu.VMEM((NCHUNK, CHUNK), jnp.int32),
        pltpu.VMEM((NCHUNK, CHUNK), jnp.int32),
        pltpu.VMEM((VOCAB,), jnp.float32),
        pltpu.VMEM((CHUNK, VOCAB), jnp.float32),
        pltpu.VMEM((16,), jnp.float32),
        pltpu.SemaphoreType.DMA,
    ),
)
def _sc_gather(table, idx3, tgt3, lse, out, partials,
               idx_v, tgt_v, lse_v, rows_v, acc_v, gsem):
    wid = lax.axis_index("s") * NC + lax.axis_index("c")
    base = wid * PER_W
    pltpu.sync_copy(idx3.at[wid], idx_v)
    pltpu.sync_copy(tgt3.at[wid], tgt_v)
    pltpu.sync_copy(lse, lse_v)

    def chunk_body(c, acc):
        pltpu.async_copy(table.at[idx_v.at[c]], rows_v, gsem).wait()
        for g in range(CHUNK // 16):
            rid = jnp.arange(16, dtype=jnp.int32) + (g * 16)
            iv = idx_v[c, pl.ds(g * 16, 16)]
            tg = tgt_v[c, pl.ds(g * 16, 16)]
            val = plsc.load_gather(rows_v, [rid, tg])
            lsev = plsc.load_gather(lse_v, [iv])
            acc = acc + (lsev - val)
        pltpu.sync_copy(rows_v, out.at[pl.ds(base + c * CHUNK, CHUNK)])
        return acc

    acc = lax.fori_loop(0, NCHUNK, chunk_body,
                        jnp.zeros((16,), jnp.float32))
    acc_v[...] = acc
    pltpu.sync_copy(acc_v, partials.at[wid])


def kernel(idx, targets, table):
    idx3 = idx.reshape(NW, NCHUNK, CHUNK)
    tgt3 = targets.reshape(NW, NCHUNK, CHUNK)
    lse = _row_lse(table).reshape(VOCAB)
    logits, partials = _sc_gather(table, idx3, tgt3, lse)
    loss = _finalize(partials)[0, 0]
    return (logits, loss)


# speedup vs baseline: 1.6105x; 1.6105x over previous
import functools

import jax
import jax.numpy as jnp
from jax import lax
from jax.experimental import pallas as pl
from jax.experimental.pallas import tpu as pltpu
from jax.experimental.pallas import tpu_sc as plsc

VOCAB = 1000
B, T = 1024, 50
N = B * T
NC, NS = 2, 16
NW = NC * NS
PER_W = N // NW
CHUNK = 32
NCHUNK = PER_W // CHUNK


def _lse_body(table_ref, lse_ref):
    x = table_ref[...]
    m = jnp.max(x, axis=1, keepdims=True)
    s = jnp.sum(jnp.exp(x - m), axis=1, keepdims=True)
    lse_ref[...] = m + jnp.log(s)


def _row_lse(table):
    return pl.pallas_call(
        _lse_body,
        out_shape=jax.ShapeDtypeStruct((VOCAB, 1), jnp.float32),
    )(table)


def _fin_body(p_ref, loss_ref):
    loss_ref[...] = jnp.sum(p_ref[...], keepdims=True) * (1.0 / N)


def _finalize(partials):
    return pl.pallas_call(
        _fin_body,
        out_shape=jax.ShapeDtypeStruct((1, 1), jnp.float32),
    )(partials)


_mesh = plsc.VectorSubcoreMesh(core_axis_name="c", subcore_axis_name="s")


@functools.partial(
    pl.kernel,
    out_type=(
        jax.ShapeDtypeStruct((N, VOCAB), jnp.float32),
        jax.ShapeDtypeStruct((NW, 16), jnp.float32),
    ),
    mesh=_mesh,
    compiler_params=pltpu.CompilerParams(
        needs_layout_passes=False, use_tc_tiling_on_sc=False),
    scratch_types=(
        pltpu.VMEM((NCHUNK, CHUNK), jnp.int32),
        pltpu.VMEM((NCHUNK, CHUNK), jnp.int32),
        pltpu.VMEM((VOCAB,), jnp.float32),
        pltpu.VMEM((CHUNK, VOCAB), jnp.float32),
        pltpu.VMEM((16,), jnp.float32),
        pltpu.SemaphoreType.DMA,
    ),
)
def _sc_gather(table, idx3, tgt3, lse, out, partials,
               idx_v, tgt_v, lse_v, rows_v, acc_v, gsem):
    wid = lax.axis_index("s") * NC + lax.axis_index("c")
    base = wid * PER_W
    pltpu.sync_copy(idx3.at[wid], idx_v)
    pltpu.sync_copy(tgt3.at[wid], tgt_v)
    pltpu.sync_copy(lse, lse_v)

    def chunk_body(c, acc):
        pltpu.async_copy(table.at[idx_v.at[c]], rows_v, gsem).wait()
        for g in range(CHUNK // 16):
            rid = jnp.arange(16, dtype=jnp.int32) + (g * 16)
            iv = idx_v[c, pl.ds(g * 16, 16)]
            tg = tgt_v[c, pl.ds(g * 16, 16)]
            val = plsc.load_gather(rows_v, [rid, tg])
            lsev = plsc.load_gather(lse_v, [iv])
            acc = acc + (lsev - val)
        pltpu.sync_copy(rows_v, out.at[pl.ds(base + c * CHUNK, CHUNK)])
        return acc

    acc = lax.fori_loop(0, NCHUNK, chunk_body,
                        jnp.zeros((16,), jnp.float32))
    acc_v[...] = acc
    pltpu.sync_copy(acc_v, partials.at[wid])


def kernel(idx, targets, table):
    idx3 = idx.reshape(NW, NCHUNK, CHUNK)
    tgt3 = targets.reshape(NW, NCHUNK, CHUNK)
    lse = _row_lse(table).reshape(VOCAB)
    logits, partials = _sc_gather(table, idx3, tgt3, lse)
    loss = _finalize(partials)[0, 0]
    return (logits, loss)


# speedup vs baseline: 1.6956x; 1.0529x over previous
import functools

import jax
import jax.numpy as jnp
from jax import lax
from jax.experimental import pallas as pl
from jax.experimental.pallas import tpu as pltpu
from jax.experimental.pallas import tpu_sc as plsc

VOCAB = 1000
B, T = 1024, 50
N = B * T
NC, NS = 2, 16
NW = NC * NS
PER_W = N // NW
CHUNK = 32
NCHUNK = PER_W // CHUNK


def _lse_body(table_ref, lse_ref):
    x = table_ref[...]
    m = jnp.max(x, axis=1, keepdims=True)
    s = jnp.sum(jnp.exp(x - m), axis=1, keepdims=True)
    lse_ref[...] = m + jnp.log(s)


def _row_lse(table):
    return pl.pallas_call(
        _lse_body,
        out_shape=jax.ShapeDtypeStruct((VOCAB, 1), jnp.float32),
    )(table)


def _fin_body(p_ref, loss_ref):
    loss_ref[...] = jnp.sum(p_ref[...], keepdims=True) * (1.0 / N)


def _finalize(partials):
    return pl.pallas_call(
        _fin_body,
        out_shape=jax.ShapeDtypeStruct((1, 1), jnp.float32),
    )(partials)


_mesh = plsc.VectorSubcoreMesh(core_axis_name="c", subcore_axis_name="s")


@functools.partial(
    pl.kernel,
    out_type=(
        jax.ShapeDtypeStruct((N, VOCAB), jnp.float32),
        jax.ShapeDtypeStruct((NW, 16), jnp.float32),
    ),
    mesh=_mesh,
    compiler_params=pltpu.CompilerParams(
        needs_layout_passes=False, use_tc_tiling_on_sc=False),
    scratch_types=(
        pltpu.VMEM((NCHUNK, CHUNK), jnp.int32),
        pltpu.VMEM((NCHUNK, CHUNK), jnp.int32),
        pltpu.VMEM((VOCAB,), jnp.float32),
        pltpu.VMEM((2, CHUNK, VOCAB), jnp.float32),
        pltpu.VMEM((16,), jnp.float32),
        pltpu.SemaphoreType.DMA,
        pltpu.SemaphoreType.DMA,
        pltpu.SemaphoreType.DMA,
        pltpu.SemaphoreType.DMA,
    ),
)
def _sc_gather(table, idx3, tgt3, lse, out, partials,
               idx_v, tgt_v, lse_v, rows2, acc_v,
               gsem0, gsem1, osem0, osem1):
    wid = lax.axis_index("s") * NC + lax.axis_index("c")
    base = wid * PER_W
    pltpu.sync_copy(idx3.at[wid], idx_v)
    pltpu.sync_copy(tgt3.at[wid], tgt_v)
    pltpu.sync_copy(lse, lse_v)

    buf0, buf1 = rows2.at[0], rows2.at[1]

    def gather(c, buf, sem):
        pltpu.async_copy(table.at[idx_v.at[c]], buf, sem)

    def gather_wait(buf, sem):
        pltpu.make_async_copy(table.at[idx_v.at[0]], buf, sem).wait()

    def outcopy(c, buf, sem):
        pltpu.async_copy(buf, out.at[pl.ds(base + c * CHUNK, CHUNK)], sem)

    def outcopy_wait(buf, sem):
        pltpu.make_async_copy(buf, out.at[pl.ds(base, CHUNK)], sem).wait()

    def nll(c, buf, acc):
        for g in range(CHUNK // 16):
            rid = jnp.arange(16, dtype=jnp.int32) + (g * 16)
            iv = idx_v[c, pl.ds(g * 16, 16)]
            tg = tgt_v[c, pl.ds(g * 16, 16)]
            val = plsc.load_gather(buf, [rid, tg])
            lsev = plsc.load_gather(lse_v, [iv])
            acc = acc + (lsev - val)
        return acc

    gather(0, buf0, gsem0)

    def pair_body(i, acc):
        c0 = 2 * i
        c1 = c0 + 1

        @pl.when(i > 0)
        def _():
            outcopy_wait(buf1, osem1)

        gather(c1, buf1, gsem1)
        gather_wait(buf0, gsem0)
        acc = nll(c0, buf0, acc)
        outcopy(c0, buf0, osem0)
        gather_wait(buf1, gsem1)
        acc = nll(c1, buf1, acc)

        @pl.when(i < NCHUNK // 2 - 1)
        def _():
            outcopy_wait(buf0, osem0)
            gather(c0 + 2, buf0, gsem0)

        outcopy(c1, buf1, osem1)
        return acc

    acc = lax.fori_loop(0, NCHUNK // 2, pair_body,
                        jnp.zeros((16,), jnp.float32))
    outcopy_wait(buf0, osem0)
    outcopy_wait(buf1, osem1)
    acc_v[...] = acc
    pltpu.sync_copy(acc_v, partials.at[wid])


def kernel(idx, targets, table):
    idx3 = idx.reshape(NW, NCHUNK, CHUNK)
    tgt3 = targets.reshape(NW, NCHUNK, CHUNK)
    lse = _row_lse(table).reshape(VOCAB)
    logits, partials = _sc_gather(table, idx3, tgt3, lse)
    loss = _finalize(partials)[0, 0]
    return (logits, loss)


# speedup vs baseline: 2.5062x; 1.4780x over previous
import functools

import jax
import jax.numpy as jnp
from jax import lax
from jax.experimental import pallas as pl
from jax.experimental.pallas import tpu as pltpu
from jax.experimental.pallas import tpu_sc as plsc

VOCAB = 1000
VPAD = 1024
NCT = VPAD // 128
B, T = 1024, 50
N = B * T
NC, NS = 2, 16
NW = NC * NS
PER_W = N // NW
CHUNK = 32
NCHUNK = PER_W // CHUNK
EDGE = VOCAB - 7 * 128


def _lse_body(table_ref, lse_ref):
    x = table_ref[...]
    m = jnp.max(x, axis=1, keepdims=True)
    s = jnp.sum(jnp.exp(x - m), axis=1, keepdims=True)
    lse_ref[...] = m + jnp.log(s)


def _row_lse(table):
    return pl.pallas_call(
        _lse_body,
        out_shape=jax.ShapeDtypeStruct((VOCAB, 1), jnp.float32),
    )(table)


def _fin_body(p_ref, loss_ref):
    loss_ref[...] = jnp.sum(p_ref[...], keepdims=True) * (1.0 / N)


def _finalize(partials):
    return pl.pallas_call(
        _fin_body,
        out_shape=jax.ShapeDtypeStruct((1, 1), jnp.float32),
    )(partials)


_mesh = plsc.VectorSubcoreMesh(core_axis_name="c", subcore_axis_name="s")


@functools.partial(
    pl.kernel,
    out_type=(
        jax.ShapeDtypeStruct((N, VOCAB), jnp.float32),
        jax.ShapeDtypeStruct((NW, 16), jnp.float32),
    ),
    mesh=_mesh,
    compiler_params=pltpu.CompilerParams(
        needs_layout_passes=False, use_tc_tiling_on_sc=True),
    scratch_types=(
        pltpu.VMEM((NCHUNK, CHUNK), jnp.int32),
        pltpu.VMEM((NCHUNK, CHUNK), jnp.int32),
        pltpu.VMEM((VOCAB,), jnp.float32),
        pltpu.VMEM((2, CHUNK, VOCAB), jnp.float32),
        pltpu.VMEM((2, CHUNK, 128), jnp.float32),
        pltpu.VMEM((16,), jnp.float32),
        pltpu.SemaphoreType.DMA,
        pltpu.SemaphoreType.DMA,
        pltpu.SemaphoreType.DMA,
        pltpu.SemaphoreType.DMA,
    ),
)
def _sc_gather(tablec, idx3, tgt3, lse, out, partials,
               idx_v, tgt_v, lse_v, rows2, edge2, acc_v,
               gsem0, gsem1, osem0, osem1):
    wid = lax.axis_index("s") * NC + lax.axis_index("c")
    base = wid * PER_W
    pltpu.sync_copy(idx3.at[wid], idx_v)
    pltpu.sync_copy(tgt3.at[wid], tgt_v)
    pltpu.sync_copy(lse, lse_v)

    def gather(c, b):
        ii = idx_v.at[c]
        sem = gsem0 if b == 0 else gsem1
        for ct in range(7):
            pltpu.async_copy(tablec.at[ct].at[ii],
                             rows2.at[b].at[:, pl.ds(128 * ct, 128)], sem)
        pltpu.async_copy(tablec.at[7].at[ii], edge2.at[b], sem)

    def gather_wait(b):
        sem = gsem0 if b == 0 else gsem1
        for _ in range(7):
            pltpu.make_async_copy(tablec.at[0].at[idx_v.at[0]],
                                  rows2.at[b].at[:, pl.ds(0, 128)],
                                  sem).wait()
        pltpu.make_async_copy(tablec.at[7].at[idx_v.at[0]], edge2.at[b],
                              sem).wait()

    def edge_fix(b):
        src = edge2.at[b]
        dst = rows2.at[b]
        for j in range(CHUNK):
            for m in (0, 16, 32, 48, 64, 80, 88):
                dst[j, pl.ds(896 + m, 16)] = src[j, pl.ds(m, 16)]

    def outcopy(c, b):
        sem = osem0 if b == 0 else osem1
        pltpu.async_copy(rows2.at[b],
                         out.at[pl.ds(base + c * CHUNK, CHUNK)], sem)

    def outcopy_wait(b):
        sem = osem0 if b == 0 else osem1
        pltpu.make_async_copy(rows2.at[b], out.at[pl.ds(base, CHUNK)],
                              sem).wait()

    def nll(c, b, acc):
        for g in range(CHUNK // 16):
            rid = jnp.arange(16, dtype=jnp.int32) + (g * 16)
            iv = idx_v[c, pl.ds(g * 16, 16)]
            tg = tgt_v[c, pl.ds(g * 16, 16)]
            val = plsc.load_gather(rows2.at[b], [rid, tg])
            lsev = plsc.load_gather(lse_v, [iv])
            acc = acc + (lsev - val)
        return acc

    gather(0, 0)

    def pair_body(i, acc):
        c0 = 2 * i
        c1 = c0 + 1

        @pl.when(i > 0)
        def _():
            outcopy_wait(1)

        gather(c1, 1)
        gather_wait(0)
        edge_fix(0)
        acc = nll(c0, 0, acc)
        outcopy(c0, 0)
        gather_wait(1)
        edge_fix(1)
        acc = nll(c1, 1, acc)

        @pl.when(i < NCHUNK // 2 - 1)
        def _():
            outcopy_wait(0)
            gather(c0 + 2, 0)

        outcopy(c1, 1)
        return acc

    acc = lax.fori_loop(0, NCHUNK // 2, pair_body,
                        jnp.zeros((16,), jnp.float32))
    outcopy_wait(0)
    outcopy_wait(1)
    acc_v[...] = acc
    pltpu.sync_copy(acc_v, partials.at[wid])


def kernel(idx, targets, table):
    idx3 = idx.reshape(NW, NCHUNK, CHUNK)
    tgt3 = targets.reshape(NW, NCHUNK, CHUNK)
    lse = _row_lse(table).reshape(VOCAB)
    table_p = jnp.pad(table, ((0, 0), (0, VPAD - VOCAB)))
    tablec = table_p.reshape(VOCAB, NCT, 128).transpose(1, 0, 2)
    logits, partials = _sc_gather(tablec, idx3, tgt3, lse)
    loss = _finalize(partials)[0, 0]
    return (logits, loss)
